# baseline (speedup 1.0000x reference)
import jax
import jax.numpy as jnp
from jax import lax
from jax.experimental import pallas as pl

FB = 128


def _build_enc_kernel(ti_ref, wft_ref, wtime_ref, wtk_ref, ef_ref, eb_ref):
    feature_types, third = wft_ref.shape
    n_tickers = wtk_ref.shape[0]
    max_time = wtime_ref.shape[0]
    num_features = ef_ref.shape[0]
    batch = eb_ref.shape[0]
    d_model = 3 * third
    hi = lax.Precision.HIGHEST

    f = lax.broadcasted_iota(jnp.int32, (num_features, 1), 0)
    g = lax.broadcasted_iota(jnp.int32, (1, feature_types), 1)
    t = lax.broadcasted_iota(jnp.int32, (1, n_tickers), 1)
    sel_ft = (f // n_tickers == g).astype(jnp.float32)
    sel_tk = (f % n_tickers == t).astype(jnp.float32)
    ftrows = jnp.dot(sel_ft, wft_ref[...], precision=hi)
    tkrows = jnp.dot(sel_tk, wtk_ref[...], precision=hi)

    col = lax.broadcasted_iota(jnp.int32, (third, d_model), 1)
    row = lax.broadcasted_iota(jnp.int32, (third, d_model), 0)
    base = col - 3 * row
    p0 = (base == 0).astype(jnp.float32)
    p1 = (base == 1).astype(jnp.float32)
    p2 = (base == 2).astype(jnp.float32)

    ef_ref[...] = (jnp.dot(ftrows, p0, precision=hi)
                   + jnp.dot(tkrows, p2, precision=hi))

    t_iota = lax.broadcasted_iota(jnp.int32, (batch, max_time), 1)
    onehot = (t_iota == ti_ref[...]).astype(jnp.float32)
    tm = jnp.dot(onehot, wtime_ref[...], precision=hi)
    eb_ref[...] = jnp.dot(tm, p1, precision=hi)


def _add_kernel(x_ref, ef_ref, eb_ref, o_ref):
    o_ref[...] = x_ref[...] + ef_ref[...][:, None, :] + eb_ref[...][None, :, :]


@jax.jit
def kernel(x, time_indices, W_ft, W_time, W_tk):
    num_features, batch, d_model = x.shape
    feature_types, third = W_ft.shape
    n_tickers = W_tk.shape[0]
    max_time = W_time.shape[0]
    ti = time_indices.astype(jnp.int32).reshape(batch, 1)

    e_f, e_b = pl.pallas_call(
        _build_enc_kernel,
        out_shape=(
            jax.ShapeDtypeStruct((num_features, d_model), x.dtype),
            jax.ShapeDtypeStruct((batch, d_model), x.dtype),
        ),
    )(ti, W_ft, W_time, W_tk)

    grid = (num_features // FB,)
    return pl.pallas_call(
        _add_kernel,
        grid=grid,
        in_specs=[
            pl.BlockSpec((FB, batch, d_model), lambda i: (i, 0, 0)),
            pl.BlockSpec((FB, d_model), lambda i: (i, 0)),
            pl.BlockSpec((batch, d_model), lambda i: (0, 0)),
        ],
        out_specs=pl.BlockSpec((FB, batch, d_model), lambda i: (i, 0, 0)),
        out_shape=jax.ShapeDtypeStruct(x.shape, x.dtype),
    )(x, e_f, e_b)


# speedup vs baseline: 1.1816x; 1.1816x over previous
import jax
import jax.numpy as jnp
from jax import lax
from jax.experimental import pallas as pl

FB = 128


def _enc_add_kernel(ti_ref, x_ref, wft_ref, wtime_ref, wtk_ref, o_ref):
    i = pl.program_id(0)
    third = wft_ref.shape[1]
    d_model = 3 * third
    batch = x_ref.shape[1]
    max_time = wtime_ref.shape[0]

    col = lax.broadcasted_iota(jnp.int32, (third, d_model), 1)
    row = lax.broadcasted_iota(jnp.int32, (third, d_model), 0)
    base = col - 3 * row
    p0 = (base == 0).astype(jnp.float32)
    p1 = (base == 1).astype(jnp.float32)
    p2 = (base == 2).astype(jnp.float32)

    ft = wft_ref[pl.ds(i, 1), :]
    tk = wtk_ref[...]
    t_iota = lax.broadcasted_iota(jnp.int32, (batch, max_time), 1)
    onehot = (t_iota == ti_ref[...]).astype(jnp.float32)
    tm = jnp.dot(onehot, wtime_ref[...], preferred_element_type=jnp.float32)

    e_f = (jnp.dot(ft, p0, preferred_element_type=jnp.float32)
           + jnp.dot(tk, p2, preferred_element_type=jnp.float32))
    e_b = jnp.dot(tm, p1, preferred_element_type=jnp.float32)

    o_ref[...] = x_ref[...] + e_f[:, None, :] + e_b[None, :, :]


@jax.jit
def kernel(x, time_indices, W_ft, W_time, W_tk):
    num_features, batch, d_model = x.shape
    feature_types, third = W_ft.shape
    n_tickers = W_tk.shape[0]
    max_time = W_time.shape[0]
    ti = time_indices.astype(jnp.int32).reshape(batch, 1)

    nj = n_tickers // FB
    grid = (feature_types, nj)
    return pl.pallas_call(
        _enc_add_kernel,
        grid=grid,
        in_specs=[
            pl.BlockSpec((batch, 1), lambda i, j: (0, 0)),
            pl.BlockSpec((FB, batch, d_model), lambda i, j: (i * nj + j, 0, 0)),
            pl.BlockSpec((feature_types, third), lambda i, j: (0, 0)),
            pl.BlockSpec((max_time, third), lambda i, j: (0, 0)),
            pl.BlockSpec((FB, third), lambda i, j: (j, 0)),
        ],
        out_specs=pl.BlockSpec((FB, batch, d_model),
                               lambda i, j: (i * nj + j, 0, 0)),
        out_shape=jax.ShapeDtypeStruct(x.shape, x.dtype),
    )(ti, x, W_ft, W_time, W_tk)


# speedup vs baseline: 1.1961x; 1.0123x over previous
import jax
import jax.numpy as jnp
from jax import lax
from jax.experimental import pallas as pl
from jax.experimental.pallas import tpu as pltpu

CH = 16
NBUF = 8


def _pipeline_kernel(ti_ref, wft_ref, wtime_ref, wtk_ref, x_ref, o_ref,
                     in_buf, out_buf, ef_ref, eb_ref, in_sem, out_sem):
    num_features, batch, d_model = x_ref.shape
    feature_types, third = wft_ref.shape
    n_tickers = wtk_ref.shape[0]
    max_time = wtime_ref.shape[0]
    n_chunks = num_features // CH

    def in_copy(c, b):
        return pltpu.make_async_copy(
            x_ref.at[pl.ds(c * CH, CH)], in_buf.at[b], in_sem.at[b])

    def out_copy(c, b):
        return pltpu.make_async_copy(
            out_buf.at[b], o_ref.at[pl.ds(c * CH, CH)], out_sem.at[b])

    for b in range(NBUF):
        in_copy(b, b).start()

    f = lax.broadcasted_iota(jnp.int32, (num_features, 1), 0)
    g = lax.broadcasted_iota(jnp.int32, (1, feature_types), 1)
    t = lax.broadcasted_iota(jnp.int32, (1, n_tickers), 1)
    sel_ft = (f // n_tickers == g).astype(jnp.float32)
    sel_tk = (f % n_tickers == t).astype(jnp.float32)
    ftrows = jnp.dot(sel_ft, wft_ref[...], preferred_element_type=jnp.float32)
    tkrows = jnp.dot(sel_tk, wtk_ref[...], preferred_element_type=jnp.float32)

    col = lax.broadcasted_iota(jnp.int32, (third, d_model), 1)
    row = lax.broadcasted_iota(jnp.int32, (third, d_model), 0)
    base = col - 3 * row
    p0 = (base == 0).astype(jnp.float32)
    p1 = (base == 1).astype(jnp.float32)
    p2 = (base == 2).astype(jnp.float32)
    ef_ref[...] = (jnp.dot(ftrows, p0, preferred_element_type=jnp.float32)
                   + jnp.dot(tkrows, p2, preferred_element_type=jnp.float32))

    t_iota = lax.broadcasted_iota(jnp.int32, (batch, max_time), 1)
    onehot = (t_iota == ti_ref[...]).astype(jnp.float32)
    tm = jnp.dot(onehot, wtime_ref[...], preferred_element_type=jnp.float32)
    eb_ref[...] = jnp.dot(tm, p1, preferred_element_type=jnp.float32)

    def body(i, _):
        b = lax.rem(i, NBUF)
        in_copy(i, b).wait()

        @pl.when(i >= NBUF)
        def _():
            out_copy(i - NBUF, b).wait()

        ef = ef_ref[pl.ds(i * CH, CH), :]
        out_buf[b] = (in_buf[b] + ef[:, None, :] + eb_ref[...][None, :, :])
        out_copy(i, b).start()

        @pl.when(i + NBUF < n_chunks)
        def _():
            in_copy(i + NBUF, b).start()

        return 0

    lax.fori_loop(0, n_chunks, body, 0)

    for b in range(NBUF):
        out_copy(0, b).wait()


@jax.jit
def kernel(x, time_indices, W_ft, W_time, W_tk):
    num_features, batch, d_model = x.shape
    ti = time_indices.astype(jnp.int32).reshape(batch, 1)
    return pl.pallas_call(
        _pipeline_kernel,
        in_specs=[
            pl.BlockSpec(memory_space=pltpu.MemorySpace.VMEM),
            pl.BlockSpec(memory_space=pltpu.MemorySpace.VMEM),
            pl.BlockSpec(memory_space=pltpu.MemorySpace.VMEM),
            pl.BlockSpec(memory_space=pltpu.MemorySpace.VMEM),
            pl.BlockSpec(memory_space=pl.ANY),
        ],
        out_specs=pl.BlockSpec(memory_space=pl.ANY),
        out_shape=jax.ShapeDtypeStruct(x.shape, x.dtype),
        scratch_shapes=[
            pltpu.VMEM((NBUF, CH, batch, d_model), jnp.float32),
            pltpu.VMEM((NBUF, CH, batch, d_model), jnp.float32),
            pltpu.VMEM((num_features, d_model), jnp.float32),
            pltpu.VMEM((batch, d_model), jnp.float32),
            pltpu.SemaphoreType.DMA((NBUF,)),
            pltpu.SemaphoreType.DMA((NBUF,)),
        ],
    )(ti, W_ft, W_time, W_tk, x)


# speedup vs baseline: 1.1961x; 1.0000x over previous
import jax
import jax.numpy as jnp
from jax import lax
from jax.experimental import pallas as pl
from jax.experimental.pallas import tpu as pltpu

CH = 32
NBUF = 8


def _pipeline_kernel(ti_ref, wft_ref, wtime_ref, wtk_ref, x_ref, o_ref,
                     in_buf, out_buf, ef_ref, eb_ref, in_sem, out_sem):
    num_features, batch, d_model = x_ref.shape
    feature_types, third = wft_ref.shape
    n_tickers = wtk_ref.shape[0]
    max_time = wtime_ref.shape[0]
    n_chunks = num_features // CH

    def in_copy(c, b):
        return pltpu.make_async_copy(
            x_ref.at[pl.ds(c * CH, CH)], in_buf.at[b], in_sem.at[b])

    def out_copy(c, b):
        return pltpu.make_async_copy(
            out_buf.at[b], o_ref.at[pl.ds(c * CH, CH)], out_sem.at[b])

    for b in range(NBUF):
        in_copy(b, b).start()

    f = lax.broadcasted_iota(jnp.int32, (num_features, 1), 0)
    g = lax.broadcasted_iota(jnp.int32, (1, feature_types), 1)
    t = lax.broadcasted_iota(jnp.int32, (1, n_tickers), 1)
    sel_ft = (f // n_tickers == g).astype(jnp.float32)
    sel_tk = (f % n_tickers == t).astype(jnp.float32)
    ftrows = jnp.dot(sel_ft, wft_ref[...], preferred_element_type=jnp.float32)
    tkrows = jnp.dot(sel_tk, wtk_ref[...], preferred_element_type=jnp.float32)

    col = lax.broadcasted_iota(jnp.int32, (third, d_model), 1)
    row = lax.broadcasted_iota(jnp.int32, (third, d_model), 0)
    base = col - 3 * row
    p0 = (base == 0).astype(jnp.float32)
    p1 = (base == 1).astype(jnp.float32)
    p2 = (base == 2).astype(jnp.float32)
    ef_ref[...] = (jnp.dot(ftrows, p0, preferred_element_type=jnp.float32)
                   + jnp.dot(tkrows, p2, preferred_element_type=jnp.float32))

    t_iota = lax.broadcasted_iota(jnp.int32, (batch, max_time), 1)
    onehot = (t_iota == ti_ref[...]).astype(jnp.float32)
    tm = jnp.dot(onehot, wtime_ref[...], preferred_element_type=jnp.float32)
    eb_ref[...] = jnp.dot(tm, p1, preferred_element_type=jnp.float32)

    def body(i, _):
        b = lax.rem(i, NBUF)
        in_copy(i, b).wait()

        @pl.when(i >= NBUF)
        def _():
            out_copy(i - NBUF, b).wait()

        ef = ef_ref[pl.ds(i * CH, CH), :]
        out_buf[b] = (in_buf[b] + ef[:, None, :] + eb_ref[...][None, :, :])
        out_copy(i, b).start()

        @pl.when(i + NBUF < n_chunks)
        def _():
            in_copy(i + NBUF, b).start()

        return 0

    lax.fori_loop(0, n_chunks, body, 0)

    for b in range(NBUF):
        out_copy(0, b).wait()


@jax.jit
def kernel(x, time_indices, W_ft, W_time, W_tk):
    num_features, batch, d_model = x.shape
    ti = time_indices.astype(jnp.int32).reshape(batch, 1)
    return pl.pallas_call(
        _pipeline_kernel,
        in_specs=[
            pl.BlockSpec(memory_space=pltpu.MemorySpace.VMEM),
            pl.BlockSpec(memory_space=pltpu.MemorySpace.VMEM),
            pl.BlockSpec(memory_space=pltpu.MemorySpace.VMEM),
            pl.BlockSpec(memory_space=pltpu.MemorySpace.VMEM),
            pl.BlockSpec(memory_space=pl.ANY),
        ],
        out_specs=pl.BlockSpec(memory_space=pl.ANY),
        out_shape=jax.ShapeDtypeStruct(x.shape, x.dtype),
        scratch_shapes=[
            pltpu.VMEM((NBUF, CH, batch, d_model), jnp.float32),
            pltpu.VMEM((NBUF, CH, batch, d_model), jnp.float32),
            pltpu.VMEM((num_features, d_model), jnp.float32),
            pltpu.VMEM((batch, d_model), jnp.float32),
            pltpu.SemaphoreType.DMA((NBUF,)),
            pltpu.SemaphoreType.DMA((NBUF,)),
        ],
    )(ti, W_ft, W_time, W_tk, x)
